# baseline (speedup 1.0000x reference)
import functools

import jax
import jax.numpy as jnp
from jax import lax
from jax.experimental import pallas as pl
from jax.experimental.pallas import tpu as pltpu
from jax.experimental.pallas import tpu_sc as plsc

NC = 2
NS = 16
NW = NC * NS

VOCAB = 100000
DIM = 1024
NTOK = 8192
ROWS_PER_W = NTOK // NW
G = 16
NG = ROWS_PER_W // G


def _body(idx_hbm, t0, t1, t2, o0, o1, o2, idx_v, bufa, bufb, sema, semb, semi):
    wid = lax.axis_index("s") * NC + lax.axis_index("c")
    base = wid * ROWS_PER_W
    pltpu.sync_copy(idx_hbm.at[wid], idx_v)

    def fire(tab, g, buf, sem):
        vec = idx_v[pl.ds(g * G, G)]
        for r in range(G):
            pltpu.async_copy(tab.at[pl.ds(vec[r], 1)], buf.at[pl.ds(r, 1)], sem)


    for tab, out in ((t0, o0), (t1, o1), (t2, o2)):
        def drain_store(out, g, buf, sem):
            pltpu.make_async_copy(tab.at[pl.ds(0, G)], buf, sem).wait()
            pltpu.sync_copy(buf, out.at[pl.ds(base + g * G, G)])

        fire(tab, 0, bufa, sema)

        def pair(h):
            g0 = 2 * h
            fire(tab, g0 + 1, bufb, semb)
            drain_store(out, g0, bufa, sema)

            @pl.when(g0 + 2 < NG)
            def _():
                fire(tab, g0 + 2, bufa, sema)

            drain_store(out, g0 + 1, bufb, semb)

        pl.loop(0, NG // 2)(pair)


@jax.jit
def _gather3(idx, table0, table1, table2):
    mesh = plsc.VectorSubcoreMesh(core_axis_name="c", subcore_axis_name="s")
    out = jax.ShapeDtypeStruct((NTOK, 8, 128), jnp.bfloat16)
    return pl.kernel(
        _body,
        out_type=(out, out, out),
        mesh=mesh,
        scratch_types=[
            pltpu.VMEM((ROWS_PER_W,), jnp.int32),
            pltpu.VMEM((G, 8, 128), jnp.bfloat16),
            pltpu.VMEM((G, 8, 128), jnp.bfloat16),
            pltpu.SemaphoreType.DMA,
            pltpu.SemaphoreType.DMA,
            pltpu.SemaphoreType.DMA,
        ],
    )(idx, table0, table1, table2)


def kernel(inputs, table0, table1, table2):
    B, S = inputs.shape
    idx = inputs.reshape(NW, ROWS_PER_W)
    t0 = table0.reshape(VOCAB, 8, 128)
    t1 = table1.reshape(VOCAB, 8, 128)
    t2 = table2.reshape(VOCAB, 8, 128)
    ve0, ve1, ve2 = _gather3(idx, t0, t1, t2)
    ve0 = ve0.reshape(B, S, DIM)
    ve1 = ve1.reshape(B, S, DIM)
    ve2 = ve2.reshape(B, S, DIM)
    return (ve0, ve1, ve2, ve0, ve1, ve2)


# speedup vs baseline: 4.7791x; 4.7791x over previous
import jax
import jax.numpy as jnp
from jax import lax
from jax.experimental import pallas as pl
from jax.experimental.pallas import tpu as pltpu
from jax.experimental.pallas import tpu_sc as plsc

NC = 2
NS = 16
NW = NC * NS

VOCAB = 100000
DIM = 1024
NTOK = 8192
ROWS_PER_W = NTOK // NW
G = 16
NG = ROWS_PER_W // G
NV = DIM // 16


def _body(idx_hbm, t0, t1, t2, o0, o1, o2,
          idx_v, widx, bufa, bufb, obufa, obufb, sema, semb):
    wid = lax.axis_index("s") * NC + lax.axis_index("c")
    b = wid // 8
    s0 = (wid % 8) * ROWS_PER_W
    pltpu.sync_copy(idx_hbm.at[b, pl.ds(s0, ROWS_PER_W)], idx_v)
    for i in range(ROWS_PER_W // 16):
        widx[pl.ds(i * 16, 16)] = idx_v[pl.ds(i * 16, 16)] >> 1

    ti = (t0.bitcast(jnp.int32), t1.bitcast(jnp.int32), t2.bitcast(jnp.int32))
    oi = (o0.bitcast(jnp.int32), o1.bitcast(jnp.int32), o2.bitcast(jnp.int32))
    obase = wid * (ROWS_PER_W // 2)
    bufs = (bufa, bufb)
    obufs = (obufa, obufb)
    sems = (sema, semb)

    for t in range(3):
        def fire(g, slot):
            pltpu.async_copy(
                ti[t].at[widx.at[pl.ds(g * G, G)]], bufs[slot], sems[slot]
            )

        def wait(g, slot):
            pltpu.make_async_copy(
                ti[t].at[widx.at[pl.ds(g * G, G)]], bufs[slot], sems[slot]
            ).wait()

        def deint_store(g, slot):
            wait(g, slot)
            vec = idx_v[pl.ds(g * G, G)]
            buf = bufs[slot]
            obuf = obufs[slot]
            for k in range(G // 2):
                sh0 = (vec[2 * k] & 1) * 16
                sh1 = (vec[2 * k + 1] & 1) * 16

                @pl.loop(0, NV // 8)
                def _(c):
                    for u in range(8):
                        col = pl.ds((c * 8 + u) * 16, 16)
                        a = buf[2 * k, col]
                        bb = buf[2 * k + 1, col]
                        lo = (a >> sh0) & jnp.int32(0xFFFF)
                        hi = (bb >> sh1) & jnp.int32(0xFFFF)
                        obuf[k, col] = lo | (hi << 16)

            pltpu.sync_copy(obuf, oi[t].at[pl.ds(obase + g * (G // 2), G // 2)])

        fire(0, 0)

        @pl.loop(0, NG // 2)
        def _(h):
            g0 = 2 * h
            fire(g0 + 1, 1)
            deint_store(g0, 0)

            @pl.when(g0 + 2 < NG)
            def _():
                fire(g0 + 2, 0)

            deint_store(g0 + 1, 1)


@jax.jit
def _gather3(idx, table0, table1, table2):
    mesh = plsc.VectorSubcoreMesh(core_axis_name="c", subcore_axis_name="s")
    out = jax.ShapeDtypeStruct((NTOK, DIM), jnp.bfloat16)
    return pl.kernel(
        _body,
        out_type=(out, out, out),
        mesh=mesh,
        scratch_types=[
            pltpu.VMEM((ROWS_PER_W,), jnp.int32),
            pltpu.VMEM((ROWS_PER_W,), jnp.int32),
            pltpu.VMEM((G, DIM), jnp.int32),
            pltpu.VMEM((G, DIM), jnp.int32),
            pltpu.VMEM((G // 2, DIM), jnp.int32),
            pltpu.VMEM((G // 2, DIM), jnp.int32),
            pltpu.SemaphoreType.DMA,
            pltpu.SemaphoreType.DMA,
        ],
    )(idx, table0, table1, table2)


def kernel(inputs, table0, table1, table2):
    B, S = inputs.shape
    ve0, ve1, ve2 = _gather3(inputs, table0, table1, table2)
    ve0 = ve0.reshape(B, S, DIM)
    ve1 = ve1.reshape(B, S, DIM)
    ve2 = ve2.reshape(B, S, DIM)
    return (ve0, ve1, ve2, ve0, ve1, ve2)


# speedup vs baseline: 5.5437x; 1.1600x over previous
import jax
import jax.numpy as jnp
from jax import lax
from jax.experimental import pallas as pl
from jax.experimental.pallas import tpu as pltpu
from jax.experimental.pallas import tpu_sc as plsc

NC = 2
NS = 16
NW = NC * NS

VOCAB = 100000
DIM = 1024
NTOK = 8192
ROWS_PER_W = NTOK // NW
G = 16
NG = ROWS_PER_W // G
NV = DIM // 16


def _body(idx_hbm, t0, t1, t2, o0, o1, o2, o3, o4, o5,
          idx_v, widx, bufa, bufb, obufa, obufb, sema, semb):
    wid = lax.axis_index("s") * NC + lax.axis_index("c")
    b = wid // 8
    s0 = (wid % 8) * ROWS_PER_W
    pltpu.sync_copy(idx_hbm.at[b, pl.ds(s0, ROWS_PER_W)], idx_v)
    for i in range(ROWS_PER_W // 16):
        widx[pl.ds(i * 16, 16)] = idx_v[pl.ds(i * 16, 16)] >> 1

    ti = (t0.bitcast(jnp.int32), t1.bitcast(jnp.int32), t2.bitcast(jnp.int32))
    oi = (o0.bitcast(jnp.int32), o1.bitcast(jnp.int32), o2.bitcast(jnp.int32))
    oi2 = (o3.bitcast(jnp.int32), o4.bitcast(jnp.int32), o5.bitcast(jnp.int32))
    obase = wid * (ROWS_PER_W // 2)
    bufs = (bufa, bufb)
    obufs = (obufa, obufb)
    sems = (sema, semb)

    for t in range(3):
        def fire(g, slot):
            pltpu.async_copy(
                ti[t].at[widx.at[pl.ds(g * G, G)]], bufs[slot], sems[slot]
            )

        def wait(g, slot):
            pltpu.make_async_copy(
                ti[t].at[widx.at[pl.ds(g * G, G)]], bufs[slot], sems[slot]
            ).wait()

        def deint_store(g, slot):
            wait(g, slot)
            vec = idx_v[pl.ds(g * G, G)]
            buf = bufs[slot]
            obuf = obufs[slot]
            for k in range(G // 2):
                sh0 = (vec[2 * k] & 1) * 16
                sh1 = (vec[2 * k + 1] & 1) * 16

                @pl.loop(0, NV // 8)
                def _(c):
                    for u in range(8):
                        col = pl.ds((c * 8 + u) * 16, 16)
                        a = buf[2 * k, col]
                        bb = buf[2 * k + 1, col]
                        lo = (a >> sh0) & jnp.int32(0xFFFF)
                        hi = (bb >> sh1) & jnp.int32(0xFFFF)
                        obuf[k, col] = lo | (hi << 16)

            orows = pl.ds(obase + g * (G // 2), G // 2)
            pltpu.sync_copy(obuf, oi[t].at[orows])
            pltpu.sync_copy(obuf, oi2[t].at[orows])

        fire(0, 0)

        @pl.loop(0, NG // 2)
        def _(h):
            g0 = 2 * h
            fire(g0 + 1, 1)
            deint_store(g0, 0)

            @pl.when(g0 + 2 < NG)
            def _():
                fire(g0 + 2, 0)

            deint_store(g0 + 1, 1)


@jax.jit
def _gather3(idx, table0, table1, table2):
    mesh = plsc.VectorSubcoreMesh(core_axis_name="c", subcore_axis_name="s")
    out = jax.ShapeDtypeStruct((NTOK, DIM), jnp.bfloat16)
    return pl.kernel(
        _body,
        out_type=(out, out, out, out, out, out),
        mesh=mesh,
        scratch_types=[
            pltpu.VMEM((ROWS_PER_W,), jnp.int32),
            pltpu.VMEM((ROWS_PER_W,), jnp.int32),
            pltpu.VMEM((G, DIM), jnp.int32),
            pltpu.VMEM((G, DIM), jnp.int32),
            pltpu.VMEM((G // 2, DIM), jnp.int32),
            pltpu.VMEM((G // 2, DIM), jnp.int32),
            pltpu.SemaphoreType.DMA,
            pltpu.SemaphoreType.DMA,
        ],
    )(idx, table0, table1, table2)


def kernel(inputs, table0, table1, table2):
    B, S = inputs.shape
    outs = _gather3(inputs, table0, table1, table2)
    return tuple(o.reshape(B, S, DIM) for o in outs)
